# baseline (speedup 1.0000x reference)
import numpy as np

import jax
import jax.numpy as jnp
from jax import lax
from jax.experimental import pallas as pl
from jax.experimental.pallas import tpu as pltpu
from jax.experimental.pallas import tpu_sc as plsc

_M32 = np.uint32(0xFFFFFFFF)


def _rotl(x, r):
    r = np.uint32(r)
    return ((x << r) | (x >> np.uint32(32 - r))) & _M32


def _threefry2x32(key0, key1, x0, x1):
    x0 = x0.astype(np.uint32).copy()
    x1 = x1.astype(np.uint32).copy()
    ks = [np.uint32(key0), np.uint32(key1),
          np.uint32(np.uint32(key0) ^ np.uint32(key1) ^ np.uint32(0x1BD11BDA))]
    rot1 = (13, 15, 26, 6)
    rot2 = (17, 29, 16, 24)
    x0 = (x0 + ks[0]) & _M32
    x1 = (x1 + ks[1]) & _M32
    for i in range(5):
        for r in (rot1 if i % 2 == 0 else rot2):
            x0 = (x0 + x1) & _M32
            x1 = _rotl(x1, r)
            x1 = x1 ^ x0
        x0 = (x0 + ks[(i + 1) % 3]) & _M32
        x1 = (x1 + ks[(i + 2) % 3] + np.uint32(i + 1)) & _M32
    return x0, x1


def _uniform_bits(key, n):
    b1, b2 = _threefry2x32(key[0], key[1],
                           np.zeros(n, np.uint32), np.arange(n, dtype=np.uint32))
    bits = b1 ^ b2
    fb = (bits >> np.uint32(9)) | np.uint32(0x3F800000)
    u = fb.view(np.float32) - np.float32(1.0)
    mn = np.float32(1e-20)
    return np.maximum(mn, u * np.float32(np.float32(1.0) - mn) + mn)


def _build_constants():
    b1, b2 = _threefry2x32(np.uint32(0), np.uint32(42),
                           np.zeros(2, np.uint32), np.arange(2, dtype=np.uint32))
    kg = (b1[0], b2[0])
    ks = (b1[1], b2[1])
    u1 = _uniform_bits(kg, 900).astype(np.float64)
    u2 = _uniform_bits(ks, 900).astype(np.float64)
    g1 = (-np.log(-np.log(u1))).astype(np.float32).reshape(100, 9)
    e2 = (1.0 / (-np.log(u2))).astype(np.float32).reshape(100, 9)

    def to_chunks(a):
        p = np.zeros((112, 9), np.float32)
        p[:100] = a
        return np.ascontiguousarray(p.reshape(7, 16, 9).transpose(0, 2, 1))

    return np.concatenate([to_chunks(g1), to_chunks(e2)], axis=1)


_CONSTS = _build_constants()

_NCHUNK = 7

_GDN = lax.GatherDimensionNumbers(
    offset_dims=(), collapsed_slice_dims=(0,), start_index_map=(0,))


def _permute(x, idx):
    return lax.gather(x, idx.reshape(16, 1), _GDN, (1,),
                      mode=lax.GatherScatterMode.PROMISE_IN_BOUNDS)


def _select_gather(vecs, srcv, lane, active):
    acc = None
    for v in active:
        g = _permute(vecs[v], lane)
        acc = g if acc is None else jnp.where(srcv == v, g, acc)
    return acc


def _deinterleave(rows):
    l16 = lax.iota(jnp.int32, 16)
    cols = []
    for j in range(9):
        pos = l16 * 9 + j
        srcv = pos >> 4
        lane = pos & 15
        active = sorted({(l * 9 + j) >> 4 for l in range(16)})
        cols.append(_select_gather(rows, srcv, lane, active))
    return cols


def _interleave9(cols):
    l16 = lax.iota(jnp.int32, 16)
    outs = []
    for i in range(9):
        base = 16 * i
        q0 = base // 9
        r = l16 + (base - 9 * q0)
        ge1 = jnp.where(r >= 9, 1, 0)
        ge2 = jnp.where(r >= 18, 1, 0)
        q = (ge1 + ge2) + q0
        rem = r - 9 * (ge1 + ge2)
        active = sorted({(base + l) % 9 for l in range(16)})
        outs.append(_select_gather(cols, rem, q, active))
    return outs


def _interleave_pair(a, b):
    l16 = lax.iota(jnp.int32, 16)
    outs = []
    for i in range(2):
        pos = l16 + 16 * i
        q = pos >> 1
        even = (pos & 1) == 0
        outs.append(jnp.where(even, _permute(a, q), _permute(b, q)))
    return outs


def _sc_body(w_hbm, c_hbm, probs_hbm, idx_hbm, w_v, c_v, p_v, i_v, sem):
    wid = lax.axis_index("s")

    @pl.when(wid < _NCHUNK)
    def _():
        cpy = pltpu.async_copy(c_hbm.at[wid], c_v, sem)

        @pl.when(wid < _NCHUNK - 1)
        def _():
            pltpu.sync_copy(w_hbm.at[pl.ds(wid * 144, 144)], w_v)

        @pl.when(wid == _NCHUNK - 1)
        def _():
            pltpu.sync_copy(w_hbm.at[pl.ds(864, 36)], w_v.at[pl.ds(0, 36)])

        cpy.wait()

        rows = [w_v[pl.ds(16 * i, 16)] for i in range(9)]
        cols = _deinterleave(rows)

        zs = [(cols[j] + c_v[j, :]) * 2.0 for j in range(9)]
        m = zs[0]
        for j in range(1, 9):
            m = jnp.maximum(m, zs[j])
        es = [jnp.exp(z - m) for z in zs]
        s = es[0]
        for j in range(1, 9):
            s = s + es[j]
        rs = 1.0 / s
        ps = [e * rs for e in es]
        keys = [(p + 1e-7) * c_v[9 + j, :] for j, p in enumerate(ps)]

        best = keys[0]
        bi = jnp.zeros((16,), jnp.int32)
        for j in range(1, 9):
            g = keys[j] > best
            best = jnp.where(g, keys[j], best)
            bi = jnp.where(g, j, bi)
        neg = jnp.full((16,), -1.0, jnp.float32)
        best2 = jnp.where(bi == 0, neg, keys[0])
        b2i = jnp.zeros((16,), jnp.int32)
        for j in range(1, 9):
            cand = jnp.where(bi == j, neg, keys[j])
            g = cand > best2
            best2 = jnp.where(g, cand, best2)
            b2i = jnp.where(g, j, b2i)

        pflat = _interleave9(ps)
        for i in range(9):
            p_v[pl.ds(16 * i, 16)] = pflat[i]
        iflat = _interleave_pair(bi, b2i)
        i_v[pl.ds(0, 16)] = iflat[0]
        i_v[pl.ds(16, 16)] = iflat[1]

        @pl.when(wid < _NCHUNK - 1)
        def _():
            pltpu.sync_copy(p_v, probs_hbm.at[pl.ds(wid * 144, 144)])
            pltpu.sync_copy(i_v, idx_hbm.at[pl.ds(wid * 32, 32)])

        @pl.when(wid == _NCHUNK - 1)
        def _():
            pltpu.sync_copy(p_v.at[pl.ds(0, 36)], probs_hbm.at[pl.ds(864, 36)])
            pltpu.sync_copy(i_v.at[pl.ds(0, 8)], idx_hbm.at[pl.ds(192, 8)])


_sc_call_cache = []


def _get_sc_call():
    if not _sc_call_cache:
        _sc_call_cache.append(pl.kernel(
            _sc_body,
            out_type=[jax.ShapeDtypeStruct((900,), jnp.float32),
                      jax.ShapeDtypeStruct((200,), jnp.int32)],
            mesh=plsc.VectorSubcoreMesh(core_axis_name="c",
                                        subcore_axis_name="s", num_cores=1,
                                        num_subcores=8),
            scratch_types=[pltpu.VMEM((144,), jnp.float32),
                           pltpu.VMEM((18, 16), jnp.float32),
                           pltpu.VMEM((144,), jnp.float32),
                           pltpu.VMEM((32,), jnp.int32),
                           pltpu.SemaphoreType.DMA],
            compiler_params=pltpu.CompilerParams(use_tc_tiling_on_sc=False),
        ))
    return _sc_call_cache[0]


def kernel(inputs, weight):
    del inputs
    probs_flat, idx_flat = _get_sc_call()(weight.reshape(900), _CONSTS)
    return probs_flat.reshape(100, 9), idx_flat.reshape(100, 2)


# speedup vs baseline: 1.0013x; 1.0013x over previous
import numpy as np

import jax
import jax.numpy as jnp
from jax import lax
from jax.experimental import pallas as pl
from jax.experimental.pallas import tpu as pltpu
from jax.experimental.pallas import tpu_sc as plsc

_M32 = np.uint32(0xFFFFFFFF)


def _rotl(x, r):
    r = np.uint32(r)
    return ((x << r) | (x >> np.uint32(32 - r))) & _M32


def _threefry2x32(key0, key1, x0, x1):
    x0 = x0.astype(np.uint32).copy()
    x1 = x1.astype(np.uint32).copy()
    ks = [np.uint32(key0), np.uint32(key1),
          np.uint32(np.uint32(key0) ^ np.uint32(key1) ^ np.uint32(0x1BD11BDA))]
    rot1 = (13, 15, 26, 6)
    rot2 = (17, 29, 16, 24)
    x0 = (x0 + ks[0]) & _M32
    x1 = (x1 + ks[1]) & _M32
    for i in range(5):
        for r in (rot1 if i % 2 == 0 else rot2):
            x0 = (x0 + x1) & _M32
            x1 = _rotl(x1, r)
            x1 = x1 ^ x0
        x0 = (x0 + ks[(i + 1) % 3]) & _M32
        x1 = (x1 + ks[(i + 2) % 3] + np.uint32(i + 1)) & _M32
    return x0, x1


def _uniform_bits(key, n):
    b1, b2 = _threefry2x32(key[0], key[1],
                           np.zeros(n, np.uint32), np.arange(n, dtype=np.uint32))
    bits = b1 ^ b2
    fb = (bits >> np.uint32(9)) | np.uint32(0x3F800000)
    u = fb.view(np.float32) - np.float32(1.0)
    mn = np.float32(1e-20)
    return np.maximum(mn, u * np.float32(np.float32(1.0) - mn) + mn)


def _build_constants():
    b1, b2 = _threefry2x32(np.uint32(0), np.uint32(42),
                           np.zeros(2, np.uint32), np.arange(2, dtype=np.uint32))
    kg = (b1[0], b2[0])
    ks = (b1[1], b2[1])
    u1 = _uniform_bits(kg, 900).astype(np.float64)
    u2 = _uniform_bits(ks, 900).astype(np.float64)
    g1 = (-np.log(-np.log(u1))).astype(np.float32).reshape(100, 9)
    e2 = (1.0 / (-np.log(u2))).astype(np.float32).reshape(100, 9)

    def to_chunks(a):
        p = np.zeros((112, 9), np.float32)
        p[:100] = a
        return np.ascontiguousarray(p.reshape(7, 16, 9).transpose(0, 2, 1))

    return np.concatenate([to_chunks(g1), to_chunks(e2)], axis=1)


_CONSTS = _build_constants()

_NCHUNK = 7

_GDN = lax.GatherDimensionNumbers(
    offset_dims=(), collapsed_slice_dims=(0,), start_index_map=(0,))


def _permute(x, idx):
    return lax.gather(x, idx.reshape(16, 1), _GDN, (1,),
                      mode=lax.GatherScatterMode.PROMISE_IN_BOUNDS)


def _select_gather(vecs, srcv, lane, active):
    acc = None
    for v in active:
        g = _permute(vecs[v], lane)
        acc = g if acc is None else jnp.where(srcv == v, g, acc)
    return acc


def _deinterleave(rows):
    l16 = lax.iota(jnp.int32, 16)
    cols = []
    for j in range(9):
        pos = l16 * 9 + j
        srcv = pos >> 4
        lane = pos & 15
        active = sorted({(l * 9 + j) >> 4 for l in range(16)})
        cols.append(_select_gather(rows, srcv, lane, active))
    return cols


def _interleave9(cols):
    l16 = lax.iota(jnp.int32, 16)
    outs = []
    for i in range(9):
        base = 16 * i
        q0 = base // 9
        r = l16 + (base - 9 * q0)
        ge1 = jnp.where(r >= 9, 1, 0)
        ge2 = jnp.where(r >= 18, 1, 0)
        q = (ge1 + ge2) + q0
        rem = r - 9 * (ge1 + ge2)
        active = sorted({(base + l) % 9 for l in range(16)})
        outs.append(_select_gather(cols, rem, q, active))
    return outs


def _interleave_pair(a, b):
    l16 = lax.iota(jnp.int32, 16)
    outs = []
    for i in range(2):
        pos = l16 + 16 * i
        q = pos >> 1
        even = (pos & 1) == 0
        outs.append(jnp.where(even, _permute(a, q), _permute(b, q)))
    return outs


def _sc_body(w_hbm, c_hbm, probs_hbm, idx_hbm, w_v, c_v, p_v, i_v,
             sem_c, sem_w):
    wid = lax.axis_index("s")

    @pl.when(wid < _NCHUNK)
    def _():
        cpy = pltpu.async_copy(c_hbm.at[wid], c_v, sem_c)

        @pl.when(wid < _NCHUNK - 1)
        def _():
            pltpu.async_copy(w_hbm.at[pl.ds(wid * 144, 144)], w_v,
                             sem_w).wait()

        @pl.when(wid == _NCHUNK - 1)
        def _():
            pltpu.async_copy(w_hbm.at[pl.ds(864, 36)], w_v.at[pl.ds(0, 36)],
                             sem_w).wait()

        rows = [w_v[pl.ds(16 * i, 16)] for i in range(9)]
        cols = _deinterleave(rows)
        cpy.wait()

        zs = [(cols[j] + c_v[j, :]) * 2.0 for j in range(9)]
        m = zs[0]
        for j in range(1, 9):
            m = jnp.maximum(m, zs[j])
        es = [jnp.exp(z - m) for z in zs]
        s = es[0]
        for j in range(1, 9):
            s = s + es[j]
        rs = 1.0 / s
        ps = [e * rs for e in es]
        keys = [(p + 1e-7) * c_v[9 + j, :] for j, p in enumerate(ps)]

        best = keys[0]
        bi = jnp.zeros((16,), jnp.int32)
        for j in range(1, 9):
            g = keys[j] > best
            best = jnp.where(g, keys[j], best)
            bi = jnp.where(g, j, bi)
        neg = jnp.full((16,), -1.0, jnp.float32)
        best2 = jnp.where(bi == 0, neg, keys[0])
        b2i = jnp.zeros((16,), jnp.int32)
        for j in range(1, 9):
            cand = jnp.where(bi == j, neg, keys[j])
            g = cand > best2
            best2 = jnp.where(g, cand, best2)
            b2i = jnp.where(g, j, b2i)

        pflat = _interleave9(ps)
        for i in range(9):
            p_v[pl.ds(16 * i, 16)] = pflat[i]
        iflat = _interleave_pair(bi, b2i)
        i_v[pl.ds(0, 16)] = iflat[0]
        i_v[pl.ds(16, 16)] = iflat[1]

        @pl.when(wid < _NCHUNK - 1)
        def _():
            po = pltpu.async_copy(p_v, probs_hbm.at[pl.ds(wid * 144, 144)],
                                  sem_w)
            io = pltpu.async_copy(i_v, idx_hbm.at[pl.ds(wid * 32, 32)], sem_c)
            po.wait()
            io.wait()

        @pl.when(wid == _NCHUNK - 1)
        def _():
            po = pltpu.async_copy(p_v.at[pl.ds(0, 36)],
                                  probs_hbm.at[pl.ds(864, 36)], sem_w)
            io = pltpu.async_copy(i_v.at[pl.ds(0, 8)],
                                  idx_hbm.at[pl.ds(192, 8)], sem_c)
            po.wait()
            io.wait()


_sc_call_cache = []


def _get_sc_call():
    if not _sc_call_cache:
        _sc_call_cache.append(pl.kernel(
            _sc_body,
            out_type=[jax.ShapeDtypeStruct((900,), jnp.float32),
                      jax.ShapeDtypeStruct((200,), jnp.int32)],
            mesh=plsc.VectorSubcoreMesh(core_axis_name="c",
                                        subcore_axis_name="s", num_cores=1,
                                        num_subcores=8),
            scratch_types=[pltpu.VMEM((144,), jnp.float32),
                           pltpu.VMEM((18, 16), jnp.float32),
                           pltpu.VMEM((144,), jnp.float32),
                           pltpu.VMEM((32,), jnp.int32),
                           pltpu.SemaphoreType.DMA,
                           pltpu.SemaphoreType.DMA],
            compiler_params=pltpu.CompilerParams(use_tc_tiling_on_sc=False),
        ))
    return _sc_call_cache[0]


def kernel(inputs, weight):
    del inputs
    probs_flat, idx_flat = _get_sc_call()(weight.reshape(900), _CONSTS)
    return probs_flat.reshape(100, 9), idx_flat.reshape(100, 2)
